# Initial kernel scaffold; baseline (speedup 1.0000x reference)
import jax
import jax.numpy as jnp
from jax.experimental import pallas as pl


def kernel(x, edge_index, W1, b1, W2, b2):
    raise NotImplementedError("write your pallas kernel here")


# speedup vs baseline: 7.2993x; 7.2993x over previous
import jax
import jax.numpy as jnp
from jax import lax
from jax.experimental import pallas as pl
from jax.experimental.pallas import tpu as pltpu
from jax.experimental.pallas import tpu_sc as plsc

_N = 10000
_E = 320000
_D = 128

_NP = 10240
_NC = 2
_NS = 16
_NW = _NC * _NS
_EPW = _E // _NW
_K = 80
_NCHUNK = _EPW // _K
_RPT = _NP // _NS
_DEG_W = 16

_HN = _NP // _NC
_ZPT = 328
_HROWS = _NS * _ZPT
_CPT = _HN // _NS
_EPT = _E // _NS
_NCHUNK2 = _EPT // _K

_mesh = plsc.VectorSubcoreMesh(core_axis_name="c", subcore_axis_name="s")


def _spmem_put(stage_v, acc_sh, s, rows):
    for k in range(_NS):
        @pl.when(s == k)
        def _():
            pltpu.sync_copy(stage_v, acc_sh.at[pl.ds(k * rows, rows)])


def _spmem_get(acc_sh, stage_v, s, rows):
    for k in range(_NS):
        @pl.when(s == k)
        def _():
            pltpu.sync_copy(acc_sh.at[pl.ds(k * rows, rows)], stage_v)


def _deg_body(dst_hbm, out_hbm, idx_v, ones_v, stage_v, acc_sh):
    c = lax.axis_index("c")
    s = lax.axis_index("s")
    wid = c * _NS + s
    base = wid * _EPW

    def fill_ones(i, carry):
        ones_v[i, :] = jnp.full((_DEG_W,), 1.0, jnp.float32)
        return carry

    lax.fori_loop(0, _K, fill_ones, 0)

    def fill_zero(i, carry):
        stage_v[i, :] = jnp.zeros((_DEG_W,), jnp.float32)
        return carry

    lax.fori_loop(0, _RPT, fill_zero, 0)

    _spmem_put(stage_v, acc_sh, s, _RPT)
    plsc.subcore_barrier()

    @pl.loop(0, _NCHUNK)
    def chunk(i):
        off = base + i * _K
        pltpu.sync_copy(dst_hbm.at[pl.ds(off, _K)], idx_v)
        pltpu.sync_copy(ones_v, acc_sh.at[idx_v], add=True)
    plsc.subcore_barrier()

    _spmem_get(acc_sh, stage_v, s, _RPT)
    pltpu.sync_copy(stage_v, out_hbm.at[pl.ds(c * _NP + s * _RPT, _RPT)])


_deg_call = pl.kernel(
    _deg_body,
    out_type=jax.ShapeDtypeStruct((_NC * _NP, _DEG_W), jnp.float32),
    mesh=_mesh,
    compiler_params=pltpu.CompilerParams(use_tc_tiling_on_sc=False),
    scratch_types=[
        pltpu.VMEM((_K,), jnp.int32),
        pltpu.VMEM((_K, _DEG_W), jnp.float32),
        pltpu.VMEM((_RPT, _DEG_W), jnp.float32),
        pltpu.VMEM_SHARED((_NP, _DEG_W), jnp.float32),
    ],
)


def _scatter_body(g_hbm, src_hbm, dst_hbm, out_hbm,
                  sidx, didx, tidx, rows_v, stage_v, sem, acc_sh):
    c = lax.axis_index("c")
    s = lax.axis_index("s")
    base = s * _EPT
    row0 = c * _HN
    trash = _HN + s

    def fill_zero(i, carry):
        r = i // 8
        col = (i % 8) * 16
        stage_v[r, pl.ds(col, 16)] = jnp.zeros((16,), jnp.float32)
        return carry

    lax.fori_loop(0, _ZPT * 8, fill_zero, 0)
    _spmem_put(stage_v, acc_sh, s, _ZPT)
    plsc.subcore_barrier()

    @pl.loop(0, _NCHUNK2)
    def chunk(i):
        off = base + i * _K
        pltpu.sync_copy(src_hbm.at[pl.ds(off, _K)], sidx)
        pltpu.sync_copy(dst_hbm.at[pl.ds(off, _K)], didx)
        for j in range(_K // 16):
            v = didx[pl.ds(j * 16, 16)] - row0
            ok = (v >= 0) & (v < _HN)
            tidx[pl.ds(j * 16, 16)] = jnp.where(ok, v, trash)
        pltpu.async_copy(g_hbm.at[sidx], rows_v, sem).wait()
        pltpu.sync_copy(rows_v, acc_sh.at[tidx], add=True)
    plsc.subcore_barrier()

    for k in range(_NS):
        @pl.when(s == k)
        def _():
            pltpu.sync_copy(acc_sh.at[pl.ds(k * _CPT, _CPT)],
                            stage_v.at[pl.ds(0, _CPT)])
    pltpu.sync_copy(stage_v.at[pl.ds(0, _CPT)],
                    out_hbm.at[pl.ds(c * _HN + s * _CPT, _CPT)])


_scatter_call = pl.kernel(
    _scatter_body,
    out_type=jax.ShapeDtypeStruct((_NP, _D), jnp.float32),
    mesh=_mesh,
    compiler_params=pltpu.CompilerParams(use_tc_tiling_on_sc=False),
    scratch_types=[
        pltpu.VMEM((_K,), jnp.int32),
        pltpu.VMEM((_K,), jnp.int32),
        pltpu.VMEM((_K,), jnp.int32),
        pltpu.VMEM((_K, _D), jnp.float32),
        pltpu.VMEM((_ZPT, _D), jnp.float32),
        pltpu.SemaphoreType.DMA,
        pltpu.VMEM_SHARED((_HROWS, _D), jnp.float32),
    ],
)


def _dinv_from_parts(parts):
    deg = parts[0, :, 0:1] + parts[1, :, 0:1] + 1.0
    return lax.rsqrt(deg)


def _mm(a, w):
    return jnp.dot(a, w, preferred_element_type=jnp.float32,
                   precision=lax.Precision.HIGHEST)


def _tc_first_body(parts_ref, x_ref, w1_ref, g1_ref):
    dinv = _dinv_from_parts(parts_ref[...])
    g1_ref[...] = _mm(x_ref[...], w1_ref[...]) * dinv


def _tc_mid_body(parts_ref, acc_ref, g1_ref, w2_ref, b1_ref, g2_ref):
    dinv = _dinv_from_parts(parts_ref[...])
    acc = acc_ref[...] + g1_ref[...]
    h1 = jnp.maximum(acc * dinv + b1_ref[...][None, :], 0.0)
    g2_ref[...] = _mm(h1, w2_ref[...]) * dinv


def _tc_last_body(parts_ref, acc_ref, g2_ref, b2_ref, out_ref):
    dinv = _dinv_from_parts(parts_ref[...])
    acc = acc_ref[...] + g2_ref[...]
    out_ref[...] = acc * dinv + b2_ref[...][None, :]


_tc_first = pl.pallas_call(
    _tc_first_body,
    out_shape=jax.ShapeDtypeStruct((_NP, _D), jnp.float32),
)

_tc_mid = pl.pallas_call(
    _tc_mid_body,
    out_shape=jax.ShapeDtypeStruct((_NP, _D), jnp.float32),
)

_tc_last = pl.pallas_call(
    _tc_last_body,
    out_shape=jax.ShapeDtypeStruct((_NP, _D), jnp.float32),
)


def kernel(x, edge_index, W1, b1, W2, b2):
    src = edge_index[0]
    dst = edge_index[1]
    x_p = jnp.pad(x, ((0, _NP - _N), (0, 0)))
    deg_parts = _deg_call(dst).reshape(_NC, _NP, _DEG_W)
    g1 = _tc_first(deg_parts, x_p, W1)
    acc1 = _scatter_call(g1, src, dst)
    g2 = _tc_mid(deg_parts, acc1, g1, W2, b1)
    acc2 = _scatter_call(g2, src, dst)
    return _tc_last(deg_parts, acc2, g2, b2)[:_N]


# speedup vs baseline: 16.8107x; 2.3030x over previous
import jax
import jax.numpy as jnp
from jax import lax
from jax.experimental import pallas as pl
from jax.experimental.pallas import tpu as pltpu
from jax.experimental.pallas import tpu_sc as plsc

_N = 10000
_E = 320000
_D = 128

_NP = 10240
_NC = 2
_NS = 16
_NW = _NC * _NS
_EPW = _E // _NW
_K = 80
_NCHUNK = _EPW // _K
_RPT = _NP // _NS
_DEG_W = 16

_HN = _NP // _NC
_ZPT = 328
_HROWS = _NS * _ZPT
_CPT = _HN // _NS
_EPT = _E // _NS
_NCHUNK2 = _EPT // _K

_mesh = plsc.VectorSubcoreMesh(core_axis_name="c", subcore_axis_name="s")


def _spmem_put(stage_v, acc_sh, s, rows):
    for k in range(_NS):
        @pl.when(s == k)
        def _():
            pltpu.sync_copy(stage_v, acc_sh.at[pl.ds(k * rows, rows)])


def _spmem_get(acc_sh, stage_v, s, rows):
    for k in range(_NS):
        @pl.when(s == k)
        def _():
            pltpu.sync_copy(acc_sh.at[pl.ds(k * rows, rows)], stage_v)


def _deg_body(dst_hbm, out_hbm, idx_v, ones_v, stage_v, acc_sh):
    c = lax.axis_index("c")
    s = lax.axis_index("s")
    wid = c * _NS + s
    base = wid * _EPW

    def fill_ones(i, carry):
        ones_v[i, :] = jnp.full((_DEG_W,), 1.0, jnp.float32)
        return carry

    lax.fori_loop(0, _K, fill_ones, 0)

    def fill_zero(i, carry):
        stage_v[i, :] = jnp.zeros((_DEG_W,), jnp.float32)
        return carry

    lax.fori_loop(0, _RPT, fill_zero, 0)

    _spmem_put(stage_v, acc_sh, s, _RPT)
    plsc.subcore_barrier()

    @pl.loop(0, _NCHUNK)
    def chunk(i):
        off = base + i * _K
        pltpu.sync_copy(dst_hbm.at[pl.ds(off, _K)], idx_v)
        pltpu.sync_copy(ones_v, acc_sh.at[idx_v], add=True)
    plsc.subcore_barrier()

    _spmem_get(acc_sh, stage_v, s, _RPT)
    pltpu.sync_copy(stage_v, out_hbm.at[pl.ds(c * _NP + s * _RPT, _RPT)])


_deg_call = pl.kernel(
    _deg_body,
    out_type=jax.ShapeDtypeStruct((_NC * _NP, _DEG_W), jnp.float32),
    mesh=_mesh,
    compiler_params=pltpu.CompilerParams(use_tc_tiling_on_sc=False),
    scratch_types=[
        pltpu.VMEM((_K,), jnp.int32),
        pltpu.VMEM((_K, _DEG_W), jnp.float32),
        pltpu.VMEM((_RPT, _DEG_W), jnp.float32),
        pltpu.VMEM_SHARED((_NP, _DEG_W), jnp.float32),
    ],
)


def _scatter_body(g_hbm, src_hbm, dst_hbm, out_hbm,
                  sall, dall, tidx0, tidx1, rows0, rows1,
                  sem0, sem1, acc_sh):
    c = lax.axis_index("c")
    s = lax.axis_index("s")
    base = s * _EPT
    row0 = c * _HN
    trash = _HN + s

    def fill_zero(i, carry):
        r = i // 8
        col = (i % 8) * 16
        rows0[r, pl.ds(col, 16)] = jnp.zeros((16,), jnp.float32)
        return carry

    lax.fori_loop(0, _K * 8, fill_zero, 0)
    for k in range(_NS):
        @pl.when(s == k)
        def _():
            for j in range(4):
                pltpu.sync_copy(rows0,
                                acc_sh.at[pl.ds(k * _ZPT + j * _K, _K)])
            pltpu.sync_copy(rows0.at[pl.ds(0, 8)],
                            acc_sh.at[pl.ds(k * _ZPT + 4 * _K, 8)])

    pltpu.sync_copy(src_hbm.at[pl.ds(base, _EPT)], sall)
    pltpu.sync_copy(dst_hbm.at[pl.ds(base, _EPT)], dall)
    plsc.subcore_barrier()

    def remap(i, tidx):
        for j in range(_K // 16):
            v = dall[pl.ds(i * _K + j * 16, 16)] - row0
            ok = (v >= 0) & (v < _HN)
            tidx[pl.ds(j * 16, 16)] = jnp.where(ok, v, trash)

    def gather(i, rows_v, sem):
        pltpu.async_copy(g_hbm.at[sall.at[pl.ds(i * _K, _K)]], rows_v, sem)

    def scatter(i, rows_v, tidx, sem):
        pltpu.make_async_copy(g_hbm.at[sall.at[pl.ds(i * _K, _K)]], rows_v, sem).wait()
        remap(i, tidx)
        pltpu.sync_copy(rows_v, acc_sh.at[tidx], add=True)

    gather(0, rows0, sem0)

    @pl.loop(0, _NCHUNK2 - 2, step=2)
    def chunk(i):
        gather(i + 1, rows1, sem1)
        scatter(i, rows0, tidx0, sem0)
        gather(i + 2, rows0, sem0)
        scatter(i + 1, rows1, tidx1, sem1)

    gather(_NCHUNK2 - 1, rows1, sem1)
    scatter(_NCHUNK2 - 2, rows0, tidx0, sem0)
    scatter(_NCHUNK2 - 1, rows1, tidx1, sem1)
    plsc.subcore_barrier()

    for k in range(_NS):
        @pl.when(s == k)
        def _():
            for j in range(4):
                pltpu.sync_copy(acc_sh.at[pl.ds(k * _CPT + j * _K, _K)], rows0)
                pltpu.sync_copy(
                    rows0,
                    out_hbm.at[pl.ds(c * _HN + k * _CPT + j * _K, _K)])


_scatter_call = pl.kernel(
    _scatter_body,
    out_type=jax.ShapeDtypeStruct((_NP, _D), jnp.float32),
    mesh=_mesh,
    compiler_params=pltpu.CompilerParams(use_tc_tiling_on_sc=False),
    scratch_types=[
        pltpu.VMEM((_EPT,), jnp.int32),
        pltpu.VMEM((_EPT,), jnp.int32),
        pltpu.VMEM((_K,), jnp.int32),
        pltpu.VMEM((_K,), jnp.int32),
        pltpu.VMEM((_K, _D), jnp.float32),
        pltpu.VMEM((_K, _D), jnp.float32),
        pltpu.SemaphoreType.DMA,
        pltpu.SemaphoreType.DMA,
        pltpu.VMEM_SHARED((_HROWS, _D), jnp.float32),
    ],
)


def _dinv_from_parts(parts):
    deg = parts[0, :, 0:1] + parts[1, :, 0:1] + 1.0
    return lax.rsqrt(deg)


def _mm(a, w):
    return jnp.dot(a, w, preferred_element_type=jnp.float32,
                   precision=lax.Precision.HIGHEST)


def _tc_first_body(parts_ref, x_ref, w1_ref, g1_ref):
    dinv = _dinv_from_parts(parts_ref[...])
    g1_ref[...] = _mm(x_ref[...], w1_ref[...]) * dinv


def _tc_mid_body(parts_ref, acc_ref, g1_ref, w2_ref, b1_ref, g2_ref):
    dinv = _dinv_from_parts(parts_ref[...])
    acc = acc_ref[...] + g1_ref[...]
    h1 = jnp.maximum(acc * dinv + b1_ref[...][None, :], 0.0)
    g2_ref[...] = _mm(h1, w2_ref[...]) * dinv


def _tc_last_body(parts_ref, acc_ref, g2_ref, b2_ref, out_ref):
    dinv = _dinv_from_parts(parts_ref[...])
    acc = acc_ref[...] + g2_ref[...]
    out_ref[...] = acc * dinv + b2_ref[...][None, :]


_tc_first = pl.pallas_call(
    _tc_first_body,
    out_shape=jax.ShapeDtypeStruct((_NP, _D), jnp.float32),
)

_tc_mid = pl.pallas_call(
    _tc_mid_body,
    out_shape=jax.ShapeDtypeStruct((_NP, _D), jnp.float32),
)

_tc_last = pl.pallas_call(
    _tc_last_body,
    out_shape=jax.ShapeDtypeStruct((_NP, _D), jnp.float32),
)


def kernel(x, edge_index, W1, b1, W2, b2):
    src = edge_index[0]
    dst = edge_index[1]
    x_p = jnp.pad(x, ((0, _NP - _N), (0, 0)))
    deg_parts = _deg_call(dst).reshape(_NC, _NP, _DEG_W)
    g1 = _tc_first(deg_parts, x_p, W1)
    acc1 = _scatter_call(g1, src, dst)
    g2 = _tc_mid(deg_parts, acc1, g1, W2, b1)
    acc2 = _scatter_call(g2, src, dst)
    return _tc_last(deg_parts, acc2, g2, b2)[:_N]


# speedup vs baseline: 18.4020x; 1.0947x over previous
import jax
import jax.numpy as jnp
from jax import lax
from jax.experimental import pallas as pl
from jax.experimental.pallas import tpu as pltpu
from jax.experimental.pallas import tpu_sc as plsc

_N = 10000
_E = 320000
_D = 128

_NP = 10240
_NC = 2
_NS = 16
_NW = _NC * _NS
_EPW = _E // _NW
_K = 80
_NCHUNK = _EPW // _K
_RPT = _NP // _NS
_DEG_W = 16

_HN = _NP // _NC
_ZPT = 328
_HROWS = _NS * _ZPT
_CPT = _HN // _NS
_EPT = _E // _NS
_NCHUNK2 = _EPT // _K

_mesh = plsc.VectorSubcoreMesh(core_axis_name="c", subcore_axis_name="s")


def _spmem_put(stage_v, acc_sh, s, rows):
    for k in range(_NS):
        @pl.when(s == k)
        def _():
            pltpu.sync_copy(stage_v, acc_sh.at[pl.ds(k * rows, rows)])


def _spmem_get(acc_sh, stage_v, s, rows):
    for k in range(_NS):
        @pl.when(s == k)
        def _():
            pltpu.sync_copy(acc_sh.at[pl.ds(k * rows, rows)], stage_v)


def _deg_body(dst_hbm, out_hbm, idx_v, ones_v, stage_v, acc_sh):
    c = lax.axis_index("c")
    s = lax.axis_index("s")
    wid = c * _NS + s
    base = wid * _EPW
    pltpu.sync_copy(dst_hbm.at[pl.ds(base, _EPW)], idx_v)

    def fill_ones(i, carry):
        ones_v[i, :] = jnp.full((_DEG_W,), 1.0, jnp.float32)
        return carry

    lax.fori_loop(0, _K, fill_ones, 0)

    def fill_zero(i, carry):
        stage_v[i, :] = jnp.zeros((_DEG_W,), jnp.float32)
        return carry

    lax.fori_loop(0, _RPT, fill_zero, 0)

    _spmem_put(stage_v, acc_sh, s, _RPT)
    plsc.subcore_barrier()

    @pl.loop(0, _NCHUNK)
    def chunk(i):
        pltpu.sync_copy(ones_v, acc_sh.at[idx_v.at[pl.ds(i * _K, _K)]],
                        add=True)
    plsc.subcore_barrier()

    _spmem_get(acc_sh, stage_v, s, _RPT)
    pltpu.sync_copy(stage_v, out_hbm.at[pl.ds(c * _NP + s * _RPT, _RPT)])


_deg_call = pl.kernel(
    _deg_body,
    out_type=jax.ShapeDtypeStruct((_NC * _NP, _DEG_W), jnp.float32),
    mesh=_mesh,
    compiler_params=pltpu.CompilerParams(use_tc_tiling_on_sc=False),
    scratch_types=[
        pltpu.VMEM((_EPW,), jnp.int32),
        pltpu.VMEM((_K, _DEG_W), jnp.float32),
        pltpu.VMEM((_RPT, _DEG_W), jnp.float32),
        pltpu.VMEM_SHARED((_NP, _DEG_W), jnp.float32),
    ],
)


def _scatter_body(g_hbm, src_hbm, dst_hbm, out_hbm,
                  sall, dall, tidx0, tidx1, rows0, rows1,
                  sem0, sem1, acc_sh):
    c = lax.axis_index("c")
    s = lax.axis_index("s")
    base = s * _EPT
    row0 = c * _HN
    trash = _HN + s

    def fill_zero(i, carry):
        r = i // 8
        col = (i % 8) * 16
        rows0[r, pl.ds(col, 16)] = jnp.zeros((16,), jnp.float32)
        return carry

    lax.fori_loop(0, _K * 8, fill_zero, 0)
    for k in range(_NS):
        @pl.when(s == k)
        def _():
            for j in range(4):
                pltpu.sync_copy(rows0,
                                acc_sh.at[pl.ds(k * _ZPT + j * _K, _K)])
            pltpu.sync_copy(rows0.at[pl.ds(0, 8)],
                            acc_sh.at[pl.ds(k * _ZPT + 4 * _K, 8)])

    pltpu.sync_copy(src_hbm.at[pl.ds(base, _EPT)], sall)
    pltpu.sync_copy(dst_hbm.at[pl.ds(base, _EPT)], dall)
    plsc.subcore_barrier()

    def remap(i, tidx):
        for j in range(_K // 16):
            v = dall[pl.ds(i * _K + j * 16, 16)] - row0
            ok = (v >= 0) & (v < _HN)
            tidx[pl.ds(j * 16, 16)] = jnp.where(ok, v, trash)

    def gather(i, rows_v, sem):
        pltpu.async_copy(g_hbm.at[sall.at[pl.ds(i * _K, _K)]], rows_v, sem)

    def scatter(i, rows_v, tidx, sem):
        pltpu.make_async_copy(g_hbm.at[sall.at[pl.ds(i * _K, _K)]], rows_v, sem).wait()
        remap(i, tidx)
        pltpu.sync_copy(rows_v, acc_sh.at[tidx], add=True)

    gather(0, rows0, sem0)

    @pl.loop(0, _NCHUNK2 - 2, step=2)
    def chunk(i):
        gather(i + 1, rows1, sem1)
        scatter(i, rows0, tidx0, sem0)
        gather(i + 2, rows0, sem0)
        scatter(i + 1, rows1, tidx1, sem1)

    gather(_NCHUNK2 - 1, rows1, sem1)
    scatter(_NCHUNK2 - 2, rows0, tidx0, sem0)
    scatter(_NCHUNK2 - 1, rows1, tidx1, sem1)
    plsc.subcore_barrier()

    for k in range(_NS):
        @pl.when(s == k)
        def _():
            for j in range(4):
                pltpu.sync_copy(acc_sh.at[pl.ds(k * _CPT + j * _K, _K)], rows0)
                pltpu.sync_copy(
                    rows0,
                    out_hbm.at[pl.ds(c * _HN + k * _CPT + j * _K, _K)])


_scatter_call = pl.kernel(
    _scatter_body,
    out_type=jax.ShapeDtypeStruct((_NP, _D), jnp.float32),
    mesh=_mesh,
    compiler_params=pltpu.CompilerParams(use_tc_tiling_on_sc=False),
    scratch_types=[
        pltpu.VMEM((_EPT,), jnp.int32),
        pltpu.VMEM((_EPT,), jnp.int32),
        pltpu.VMEM((_K,), jnp.int32),
        pltpu.VMEM((_K,), jnp.int32),
        pltpu.VMEM((_K, _D), jnp.float32),
        pltpu.VMEM((_K, _D), jnp.float32),
        pltpu.SemaphoreType.DMA,
        pltpu.SemaphoreType.DMA,
        pltpu.VMEM_SHARED((_HROWS, _D), jnp.float32),
    ],
)


def _dinv_from_parts(parts):
    deg = parts[0, :, 0:1] + parts[1, :, 0:1] + 1.0
    return lax.rsqrt(deg)


def _mm(a, w):
    return jnp.dot(a, w, preferred_element_type=jnp.float32,
                   precision=lax.Precision.HIGHEST)


def _tc_first_body(parts_ref, x_ref, w1_ref, g1_ref):
    dinv = _dinv_from_parts(parts_ref[...])
    g1_ref[...] = _mm(x_ref[...], w1_ref[...]) * dinv


def _tc_mid_body(parts_ref, acc_ref, g1_ref, w2_ref, b1_ref, g2_ref):
    dinv = _dinv_from_parts(parts_ref[...])
    acc = acc_ref[...] + g1_ref[...]
    h1 = jnp.maximum(acc * dinv + b1_ref[...][None, :], 0.0)
    g2_ref[...] = _mm(h1, w2_ref[...]) * dinv


def _tc_last_body(parts_ref, acc_ref, g2_ref, b2_ref, out_ref):
    dinv = _dinv_from_parts(parts_ref[...])
    acc = acc_ref[...] + g2_ref[...]
    out_ref[...] = acc * dinv + b2_ref[...][None, :]


_tc_first = pl.pallas_call(
    _tc_first_body,
    out_shape=jax.ShapeDtypeStruct((_NP, _D), jnp.float32),
)

_tc_mid = pl.pallas_call(
    _tc_mid_body,
    out_shape=jax.ShapeDtypeStruct((_NP, _D), jnp.float32),
)

_tc_last = pl.pallas_call(
    _tc_last_body,
    out_shape=jax.ShapeDtypeStruct((_NP, _D), jnp.float32),
)


def kernel(x, edge_index, W1, b1, W2, b2):
    src = edge_index[0]
    dst = edge_index[1]
    x_p = jnp.pad(x, ((0, _NP - _N), (0, 0)))
    deg_parts = _deg_call(dst).reshape(_NC, _NP, _DEG_W)
    g1 = _tc_first(deg_parts, x_p, W1)
    acc1 = _scatter_call(g1, src, dst)
    g2 = _tc_mid(deg_parts, acc1, g1, W2, b1)
    acc2 = _scatter_call(g2, src, dst)
    return _tc_last(deg_parts, acc2, g2, b2)[:_N]
